# Initial kernel scaffold; baseline (speedup 1.0000x reference)
import jax
import jax.numpy as jnp
from jax.experimental import pallas as pl


def kernel(node_feats_real, node_feats_imag, edge_feats_real, edge_feats_imag, edge_index, Wpn, bpn, Wpe, bpe, Wn0, bn0, an0, Wnf, bnf, We0, be0, ae0, Wef, bef):
    raise NotImplementedError("write your pallas kernel here")


# speedup vs baseline: 3.1998x; 3.1998x over previous
import jax
import jax.numpy as jnp
from jax import lax
from jax.experimental import pallas as pl
from jax.experimental.pallas import tpu as pltpu
from jax.experimental.pallas import tpu_sc as plsc

NC = 2
NS = 16
NW = NC * NS

CHUNK = 80


def _proj_body(xr_ref, xi_ref, wt_ref, b_ref, out_ref):
    wt = wt_ref[...]
    b = b_ref[...]
    d = wt.shape[1]
    out_ref[:, 0:d] = jnp.dot(xr_ref[...], wt, preferred_element_type=jnp.float32) + b
    out_ref[:, d:2 * d] = jnp.dot(xi_ref[...], wt, preferred_element_type=jnp.float32) + b


def _projection(xr, xi, W, b, blk):
    n, d_in = xr.shape
    d_out = W.shape[0]
    grid = n // blk
    return pl.pallas_call(
        _proj_body,
        grid=(grid,),
        in_specs=[
            pl.BlockSpec((blk, d_in), lambda i: (i, 0)),
            pl.BlockSpec((blk, d_in), lambda i: (i, 0)),
            pl.BlockSpec((d_in, d_out), lambda i: (0, 0)),
            pl.BlockSpec((1, d_out), lambda i: (0, 0)),
        ],
        out_specs=pl.BlockSpec((blk, 2 * d_out), lambda i: (i, 0)),
        out_shape=jax.ShapeDtypeStruct((n, 2 * d_out), jnp.float32),
    )(xr, xi, W.T, b.reshape(1, d_out))


def _node_mlp_body(pn_ref, ns_ref, es_ref,
                   a_pn, a_ns, a_es, b0, alpha, wf, bf, out_ref):
    d = wf.shape[1]
    pn_full = pn_ref[...]
    ns_full = ns_ref[0] + ns_ref[1]
    es_full = es_ref[0] + es_ref[1]
    d_pn = pn_full.shape[1] // 2
    d_es = a_es.shape[0]
    for k in range(2):
        pn = pn_full[:, k * d_pn:(k + 1) * d_pn]
        nsum = ns_full[:, k * d_pn:(k + 1) * d_pn]
        esum = es_full[:, k * d_es:(k + 1) * d_es]
        h = (jnp.dot(pn, a_pn[...], preferred_element_type=jnp.float32)
             + jnp.dot(nsum, a_ns[...], preferred_element_type=jnp.float32)
             + jnp.dot(esum, a_es[...], preferred_element_type=jnp.float32)
             + b0[...])
        h = jnp.where(h >= 0, h, alpha[...] * h)
        out_ref[:, k * d:(k + 1) * d] = (
            jnp.dot(h, wf[...], preferred_element_type=jnp.float32) + bf[...])


def _node_mlp(pn_c, ns_c, es_c, Wn0, bn0, an0, Wnf, bnf, blk):
    n = pn_c.shape[0]
    d_pn = pn_c.shape[1] // 2
    h_dim = Wn0.shape[0]
    d_es = h_dim - 2 * d_pn
    d_out = Wnf.shape[0]
    grid = n // blk
    W0t = Wn0.T
    a_pn = W0t[:d_pn]
    a_ns = W0t[d_pn:2 * d_pn]
    a_es = W0t[2 * d_pn:]
    return pl.pallas_call(
        _node_mlp_body,
        grid=(grid,),
        in_specs=[
            pl.BlockSpec((blk, 2 * d_pn), lambda i: (i, 0)),
            pl.BlockSpec((2, blk, 2 * d_pn), lambda i: (0, i, 0)),
            pl.BlockSpec((2, blk, es_c.shape[2]), lambda i: (0, i, 0)),
            pl.BlockSpec((d_pn, h_dim), lambda i: (0, 0)),
            pl.BlockSpec((d_pn, h_dim), lambda i: (0, 0)),
            pl.BlockSpec((d_es, h_dim), lambda i: (0, 0)),
            pl.BlockSpec((1, h_dim), lambda i: (0, 0)),
            pl.BlockSpec((1, 1), lambda i: (0, 0)),
            pl.BlockSpec((h_dim, d_out), lambda i: (0, 0)),
            pl.BlockSpec((1, d_out), lambda i: (0, 0)),
        ],
        out_specs=pl.BlockSpec((blk, 2 * d_out), lambda i: (i, 0)),
        out_shape=jax.ShapeDtypeStruct((n, 2 * d_out), jnp.float32),
    )(pn_c, ns_c, es_c,
      a_pn, a_ns, a_es, bn0.reshape(1, h_dim), an0.reshape(1, 1),
      Wnf.T, bnf.reshape(1, d_out))


def _edge_mlp_body(pe_ref, vi_ref, vj_ref,
                   b_pe, b_vi, b_vj, b0, alpha, wf, bf, out_r, out_i):
    pe_full = pe_ref[...]
    vi_full = vi_ref[...]
    vj_full = vj_ref[...]
    d_pe = pe_full.shape[1] // 2
    d_v = vi_full.shape[1] // 2
    for k, out in enumerate((out_r, out_i)):
        pe = pe_full[:, k * d_pe:(k + 1) * d_pe]
        vi = vi_full[:, k * d_v:(k + 1) * d_v]
        vj = vj_full[:, k * d_v:(k + 1) * d_v]
        g = (jnp.dot(pe, b_pe[...], preferred_element_type=jnp.float32)
             + jnp.dot(vi, b_vi[...], preferred_element_type=jnp.float32)
             + jnp.dot(vj, b_vj[...], preferred_element_type=jnp.float32)
             + b0[...])
        g = jnp.where(g >= 0, g, alpha[...] * g)
        out[...] = jnp.dot(g, wf[...], preferred_element_type=jnp.float32) + bf[...]


def _edge_mlp(pe_c, vi_c, vj_c, We0, be0, ae0, Wef, bef, blk):
    e = pe_c.shape[0]
    d_pe = pe_c.shape[1] // 2
    d_v = vi_c.shape[1] // 2
    h_dim = We0.shape[0]
    d_out = Wef.shape[0]
    grid = e // blk
    W0t = We0.T
    b_pe = W0t[:d_pe]
    b_vi = W0t[d_pe:d_pe + d_v]
    b_vj = W0t[d_pe + d_v:]
    out_sds = jax.ShapeDtypeStruct((e, d_out), jnp.float32)
    return pl.pallas_call(
        _edge_mlp_body,
        grid=(grid,),
        in_specs=[
            pl.BlockSpec((blk, 2 * d_pe), lambda i: (i, 0)),
            pl.BlockSpec((blk, 2 * d_v), lambda i: (i, 0)),
            pl.BlockSpec((blk, 2 * d_v), lambda i: (i, 0)),
            pl.BlockSpec((d_pe, h_dim), lambda i: (0, 0)),
            pl.BlockSpec((d_v, h_dim), lambda i: (0, 0)),
            pl.BlockSpec((d_v, h_dim), lambda i: (0, 0)),
            pl.BlockSpec((1, h_dim), lambda i: (0, 0)),
            pl.BlockSpec((1, 1), lambda i: (0, 0)),
            pl.BlockSpec((h_dim, d_out), lambda i: (0, 0)),
            pl.BlockSpec((1, d_out), lambda i: (0, 0)),
        ],
        out_specs=[
            pl.BlockSpec((blk, d_out), lambda i: (i, 0)),
            pl.BlockSpec((blk, d_out), lambda i: (i, 0)),
        ],
        out_shape=[out_sds, out_sds],
    )(pe_c, vi_c, vj_c,
      b_pe, b_vi, b_vj, be0.reshape(1, h_dim), ae0.reshape(1, 1),
      Wef.T, bef.reshape(1, d_out))


def _sc_mesh():
    return plsc.VectorSubcoreMesh(core_axis_name="c", subcore_axis_name="s",
                                  num_cores=NC, num_subcores=NS)


def _node_seg_sum_sc(row3, col2, pn_c, n_pad, rows_per_sub, e):
    nchunk = row3.shape[1]
    ew = e // NW
    d_n = pn_c.shape[1]
    zrows = rows_per_sub // 8

    def body(row_hbm, col_hbm, pn_hbm, ns_hbm,
             row_v, col_v, grow0, grow1, zbuf_n, acc_n, sem0, sem1):
        cid = lax.axis_index("c")
        sid = lax.axis_index("s")
        wid = sid * NC + cid
        grows = (grow0, grow1)
        sems = (sem0, sem1)

        zero16 = jnp.zeros((16,), jnp.float32)
        for r in range(8):
            for cc in range(d_n // 16):
                zbuf_n[r, pl.ds(cc * 16, 16)] = zero16
        r0 = sid * rows_per_sub

        def zcopy(z, _):
            pltpu.sync_copy(zbuf_n, acc_n.at[pl.ds(r0 + z * 8, 8)])
            return 0

        lax.fori_loop(0, zrows, zcopy, 0)
        plsc.subcore_barrier()

        pltpu.sync_copy(row_hbm.at[wid], row_v)
        pltpu.sync_copy(col_hbm.at[wid], col_v)

        for b in range(2):
            idx = col_v.at[pl.ds(b * CHUNK, CHUNK)]
            pltpu.async_copy(pn_hbm.at[idx], grows[b], sems[b])

        def group(g, _):
            for b in range(2):
                j = g * 2 + b
                pltpu.make_async_copy(pn_hbm.at[pl.ds(0, CHUNK)],
                                      grows[b], sems[b]).wait()
                pltpu.sync_copy(grows[b], acc_n.at[row_v.at[j]], add=True)

                @pl.when(j + 2 < nchunk)
                def _prefetch():
                    idx2 = col_v.at[pl.ds((j + 2) * CHUNK, CHUNK)]
                    pltpu.async_copy(pn_hbm.at[idx2], grows[b], sems[b])

            return 0

        lax.fori_loop(0, nchunk // 2, group, 0)
        for j in range(nchunk - nchunk % 2, nchunk):
            b = j % 2
            pltpu.make_async_copy(pn_hbm.at[pl.ds(0, CHUNK)],
                                  grows[b], sems[b]).wait()
            pltpu.sync_copy(grows[b], acc_n.at[row_v.at[j]], add=True)
        plsc.subcore_barrier()

        pltpu.sync_copy(acc_n.at[pl.ds(r0, rows_per_sub)],
                        ns_hbm.at[cid, pl.ds(r0, rows_per_sub)])

    f = pl.kernel(
        body,
        out_type=jax.ShapeDtypeStruct((NC, n_pad, d_n), jnp.float32),
        mesh=_sc_mesh(),
        scratch_types=[
            pltpu.VMEM((nchunk, CHUNK), jnp.int32),
            pltpu.VMEM((ew,), jnp.int32),
            pltpu.VMEM((CHUNK, d_n), jnp.float32),
            pltpu.VMEM((CHUNK, d_n), jnp.float32),
            pltpu.VMEM((8, d_n), jnp.float32),
            pltpu.VMEM_SHARED((n_pad, d_n), jnp.float32),
            pltpu.SemaphoreType.DMA,
            pltpu.SemaphoreType.DMA,
        ],
    )
    return f(row3, col2, pn_c)


def _edge_seg_sum_sc(row3, pe_c, n_pad, rows_per_sub):
    nchunk = row3.shape[1]
    e = pe_c.shape[0]
    ew = e // NW
    d_e = pe_c.shape[1]
    d_w = 128

    def body(row_hbm, pe_hbm, es_hbm,
             row_v, pe_s, pe_v, zbuf_e, acc_e, sem):
        cid = lax.axis_index("c")
        sid = lax.axis_index("s")
        wid = sid * NC + cid

        zero16 = jnp.zeros((16,), jnp.float32)
        for r in range(8):
            for cc in range(d_w // 16):
                zbuf_e[r, pl.ds(cc * 16, 16)] = zero16
        for r in range(CHUNK):
            for cc in range(d_w // 16):
                pe_v[r, pl.ds(cc * 16, 16)] = zero16
        r0 = sid * rows_per_sub

        def zcopy(z, _):
            pltpu.sync_copy(zbuf_e, acc_e.at[pl.ds(r0 + z * 8, 8)])
            return 0

        lax.fori_loop(0, rows_per_sub // 8, zcopy, 0)
        plsc.subcore_barrier()

        pltpu.sync_copy(row_hbm.at[wid], row_v)
        ebase = wid * ew

        def step(j, _):
            pltpu.sync_copy(pe_hbm.at[pl.ds(ebase + j * CHUNK, CHUNK)], pe_s)
            for r in range(CHUNK):
                for cc in range(d_e // 16):
                    pe_v[r, pl.ds(cc * 16, 16)] = pe_s[r, pl.ds(cc * 16, 16)]
            pltpu.sync_copy(pe_v, acc_e.at[row_v.at[j]], add=True)
            return 0

        lax.fori_loop(0, nchunk, step, 0)
        plsc.subcore_barrier()

        pltpu.sync_copy(acc_e.at[pl.ds(r0, rows_per_sub)],
                        es_hbm.at[cid, pl.ds(r0, rows_per_sub)])

    f = pl.kernel(
        body,
        out_type=jax.ShapeDtypeStruct((NC, n_pad, d_w), jnp.float32),
        mesh=_sc_mesh(),
        scratch_types=[
            pltpu.VMEM((nchunk, CHUNK), jnp.int32),
            pltpu.VMEM((CHUNK, d_e), jnp.float32),
            pltpu.VMEM((CHUNK, d_w), jnp.float32),
            pltpu.VMEM((8, d_w), jnp.float32),
            pltpu.VMEM_SHARED((n_pad, d_w), jnp.float32),
            pltpu.SemaphoreType.DMA,
        ],
    )
    return f(row3, pe_c)


def _edge_gather_sc(row2, col2, no_c, e):
    ew = e // NW
    nchunk = ew // CHUNK
    d_n = no_c.shape[1]

    def body(row_hbm, col_hbm, no_hbm, vi_hbm, vj_hbm,
             row_v, col_v, a0, a1, b0, b1, sa0, sa1, sb0, sb1):
        cid = lax.axis_index("c")
        sid = lax.axis_index("s")
        wid = sid * NC + cid
        pltpu.sync_copy(row_hbm.at[wid], row_v)
        pltpu.sync_copy(col_hbm.at[wid], col_v)
        ebase = wid * ew
        abufs, bbufs = (a0, a1), (b0, b1)
        asems, bsems = (sa0, sa1), (sb0, sb1)

        def fire(j, p):
            idx_sl = pl.ds(j * CHUNK, CHUNK)
            pltpu.async_copy(no_hbm.at[row_v.at[idx_sl]], abufs[p], asems[p])
            pltpu.async_copy(no_hbm.at[col_v.at[idx_sl]], bbufs[p], bsems[p])

        def drain_and_write(j, p):
            dst = pl.ds(ebase + j * CHUNK, CHUNK)
            pltpu.make_async_copy(no_hbm.at[pl.ds(0, CHUNK)],
                                  abufs[p], asems[p]).wait()
            pltpu.sync_copy(abufs[p], vi_hbm.at[dst])
            pltpu.make_async_copy(no_hbm.at[pl.ds(0, CHUNK)],
                                  bbufs[p], bsems[p]).wait()
            pltpu.sync_copy(bbufs[p], vj_hbm.at[dst])

        for p in range(2):
            fire(p, p)

        def group(g, _):
            for p in range(2):
                j = g * 2 + p
                drain_and_write(j, p)

                @pl.when(j + 2 < nchunk)
                def _prefetch():
                    fire(j + 2, p)

            return 0

        lax.fori_loop(0, nchunk // 2, group, 0)
        for j in range(nchunk - nchunk % 2, nchunk):
            drain_and_write(j, j % 2)

    out_sds = jax.ShapeDtypeStruct((e, d_n), jnp.float32)
    f = pl.kernel(
        body,
        out_type=(out_sds, out_sds),
        mesh=_sc_mesh(),
        scratch_types=[
            pltpu.VMEM((ew,), jnp.int32),
            pltpu.VMEM((ew,), jnp.int32),
            pltpu.VMEM((CHUNK, d_n), jnp.float32),
            pltpu.VMEM((CHUNK, d_n), jnp.float32),
            pltpu.VMEM((CHUNK, d_n), jnp.float32),
            pltpu.VMEM((CHUNK, d_n), jnp.float32),
            pltpu.SemaphoreType.DMA,
            pltpu.SemaphoreType.DMA,
            pltpu.SemaphoreType.DMA,
            pltpu.SemaphoreType.DMA,
        ],
    )
    return f(row2, col2, no_c)


def kernel(node_feats_real, node_feats_imag, edge_feats_real, edge_feats_imag,
           edge_index, Wpn, bpn, Wpe, bpe, Wn0, bn0, an0, Wnf, bnf,
           We0, be0, ae0, Wef, bef):
    n = node_feats_real.shape[0]
    e = edge_feats_real.shape[0]
    d_out_node = Wnf.shape[0]

    ew = e // NW
    assert ew * NW == e and ew % CHUNK == 0
    nchunk = ew // CHUNK
    rows_per_sub = (-(-n // NS) + 7) // 8 * 8
    n_pad = rows_per_sub * NS

    row = edge_index[:, 0]
    col = edge_index[:, 1]
    row3 = row.reshape(NW, nchunk, CHUNK)
    row2 = row.reshape(NW, ew)
    col2 = col.reshape(NW, ew)

    pn_c = _projection(node_feats_real, node_feats_imag, Wpn, bpn, blk=1000)
    pe_c = _projection(edge_feats_real, edge_feats_imag, Wpe, bpe, blk=4000)

    ns_c = _node_seg_sum_sc(row3, col2, pn_c, n_pad, rows_per_sub, e)
    es_c = _edge_seg_sum_sc(row3, pe_c, n_pad, rows_per_sub)

    no_c = _node_mlp(pn_c, ns_c, es_c, Wn0, bn0, an0, Wnf, bnf, blk=1000)

    vi_c, vj_c = _edge_gather_sc(row2, col2, no_c, e)

    eo_r, eo_i = _edge_mlp(pe_c, vi_c, vj_c, We0, be0, ae0, Wef, bef, blk=4000)

    no_r = no_c[:, :d_out_node]
    no_i = no_c[:, d_out_node:]
    return no_r, no_i, eo_r, eo_i
